# Initial kernel scaffold; baseline (speedup 1.0000x reference)
import jax
import jax.numpy as jnp
from jax.experimental import pallas as pl


def kernel(h, idx):
    raise NotImplementedError("write your pallas kernel here")


# speedup vs baseline: 1.4584x; 1.4584x over previous
import functools

import jax
import jax.numpy as jnp
from jax import lax
from jax.experimental import pallas as pl
from jax.experimental.pallas import tpu as pltpu
from jax.experimental.pallas import tpu_sc as plsc

_C = 1.0
_R = 2.0
_T = 1.0
_MIN_NORM = 1e-15
_MAXNORM = (1.0 - 4e-3) / (_C ** 0.5)

_L = 16
_D = 128
_BC = 256
_NC = 2
_NS = 16
_NW = _NC * _NS


@functools.lru_cache(maxsize=None)
def _make_sc_dots(n_nodes, n_edges):
    assert n_edges % _BC == 0
    nchunk = n_edges // _BC
    nt = -(-nchunk // _NW)
    mesh = plsc.VectorSubcoreMesh(core_axis_name="c", subcore_axis_name="s")
    f32 = jnp.float32

    @functools.partial(
        pl.kernel,
        mesh=mesh,
        compiler_params=pltpu.CompilerParams(needs_layout_passes=False),
        out_type=[jax.ShapeDtypeStruct((n_edges,), f32)] * 3,
        scratch_types=[
            pltpu.VMEM((_BC // _D, _D), jnp.int32),
            pltpu.VMEM((_BC // _D, _D), jnp.int32),
            pltpu.VMEM((_BC, _D), f32),
            pltpu.VMEM((_BC, _D), f32),
            pltpu.VMEM((_BC,), f32),
            pltpu.VMEM((_BC,), f32),
            pltpu.VMEM((_BC,), f32),
            pltpu.SemaphoreType.DMA,
        ],
    )
    def sc_dots(h_hbm, i0_hbm, i1_hbm, xx_hbm, yy_hbm, xy_hbm,
                i0_v, i1_v, r0_v, r1_v, xx_v, yy_v, xy_v, sem):
        wid = lax.axis_index("s") * _NC + lax.axis_index("c")

        def do_chunk(cidx):
            pltpu.sync_copy(i0_hbm.at[cidx], i0_v)
            pltpu.sync_copy(i1_hbm.at[cidx], i1_v)
            cps = []
            for j in range(_BC // _D):
                cps.append(pltpu.async_copy(
                    h_hbm.at[i0_v.at[j]], r0_v.at[pl.ds(j * _D, _D)], sem))
                cps.append(pltpu.async_copy(
                    h_hbm.at[i1_v.at[j]], r1_v.at[pl.ds(j * _D, _D)], sem))
            for cp in cps:
                cp.wait()

            def group_body(g, carry):
                rid = g * _L + lax.iota(jnp.int32, _L)
                z = jnp.zeros((_L,), f32)

                def d_body(k, accs):
                    axx, ayy, axy = accs
                    for u in range(8):
                        d = k * 8 + u
                        col = jnp.full((_L,), d, jnp.int32)
                        x = plsc.load_gather(r0_v, [rid, col])
                        y = plsc.load_gather(r1_v, [rid, col])
                        axx = axx + x * x
                        ayy = ayy + y * y
                        axy = axy + x * y
                    return (axx, ayy, axy)

                axx, ayy, axy = lax.fori_loop(0, _D // 8, d_body, (z, z, z))
                xx_v[pl.ds(g * _L, _L)] = axx
                yy_v[pl.ds(g * _L, _L)] = ayy
                xy_v[pl.ds(g * _L, _L)] = axy
                return carry

            lax.fori_loop(0, _BC // _L, group_body, 0)
            base = cidx * _BC
            pltpu.sync_copy(xx_v, xx_hbm.at[pl.ds(base, _BC)])
            pltpu.sync_copy(yy_v, yy_hbm.at[pl.ds(base, _BC)])
            pltpu.sync_copy(xy_v, xy_hbm.at[pl.ds(base, _BC)])

        def t_body(t, carry):
            cidx = wid + t * _NW

            @pl.when(cidx < nchunk)
            def _():
                do_chunk(cidx)

            return carry

        lax.fori_loop(0, nt, t_body, 0)

    return sc_dots


def _tail_body(xx_ref, yy_ref, xy_ref, o_ref):
    xx = xx_ref[...]
    yy = yy_ref[...]
    xy = xy_ref[...]
    nx = jnp.maximum(jnp.sqrt(xx), _MIN_NORM)
    ny = jnp.maximum(jnp.sqrt(yy), _MIN_NORM)
    fx = jnp.where(nx > _MAXNORM, _MAXNORM / nx, 1.0)
    fy = jnp.where(ny > _MAXNORM, _MAXNORM / ny, 1.0)
    x2 = fx * fx * xx
    y2 = fy * fy * yy
    pxy = fx * fy * xy
    a = 1.0 - 2.0 * _C * pxy + _C * y2
    b = 1.0 - _C * x2
    num2 = a * a * x2 + b * b * y2 - 2.0 * a * b * pxy
    den = jnp.maximum(1.0 - 2.0 * _C * pxy + _C * _C * x2 * y2, _MIN_NORM)
    sqrt_c = _C ** 0.5
    z = sqrt_c * jnp.sqrt(jnp.maximum(num2, 0.0)) / den
    zc = jnp.clip(z, -1.0 + 1e-7, 1.0 - 1e-7)
    dist_c = 0.5 * jnp.log((1.0 + zc) / (1.0 - zc))
    dist = dist_c * 2.0 / sqrt_c
    sqdist = dist * dist
    o_ref[...] = 1.0 / (jnp.exp((sqdist - _R) / _T) + 1.0)


@functools.lru_cache(maxsize=None)
def _make_tail(n_edges):
    rows = n_edges // _D
    return pl.pallas_call(
        _tail_body,
        out_shape=jax.ShapeDtypeStruct((rows, _D), jnp.float32),
    )


def kernel(h, idx):
    n_nodes, d = h.shape
    n_edges = idx.shape[0]
    assert d == _D
    i0 = idx[:, 0].reshape(-1, _BC // _D, _D)
    i1 = idx[:, 1].reshape(-1, _BC // _D, _D)
    xx, yy, xy = _make_sc_dots(n_nodes, n_edges)(h, i0, i1)
    probs = _make_tail(n_edges)(
        xx.reshape(-1, _D), yy.reshape(-1, _D), xy.reshape(-1, _D))
    return probs.reshape(n_edges)


# speedup vs baseline: 3.6035x; 2.4709x over previous
import functools

import jax
import jax.numpy as jnp
from jax import lax
from jax.experimental import pallas as pl
from jax.experimental.pallas import tpu as pltpu
from jax.experimental.pallas import tpu_sc as plsc

_C = 1.0
_R = 2.0
_T = 1.0
_MIN_NORM = 1e-15
_MAXNORM = (1.0 - 4e-3) / (_C ** 0.5)

_L = 16
_D = 128
_BC = 256
_NC = 2
_NS = 16
_NW = _NC * _NS


@functools.lru_cache(maxsize=None)
def _make_sc_dots(n_nodes, n_edges):
    assert n_edges % _BC == 0
    nchunk = n_edges // _BC
    nt = -(-nchunk // _NW)
    mesh = plsc.VectorSubcoreMesh(core_axis_name="c", subcore_axis_name="s")
    f32 = jnp.float32

    @functools.partial(
        pl.kernel,
        mesh=mesh,
        compiler_params=pltpu.CompilerParams(needs_layout_passes=False),
        out_type=[jax.ShapeDtypeStruct((n_edges,), f32)] * 3,
        scratch_types=[
            pltpu.VMEM((_BC // _D, _D), jnp.int32),
            pltpu.VMEM((_BC // _D, _D), jnp.int32),
            pltpu.VMEM((_BC, _D), f32),
            pltpu.VMEM((_BC, _D), f32),
            pltpu.VMEM((_BC,), f32),
            pltpu.VMEM((_BC,), f32),
            pltpu.VMEM((_BC,), f32),
            pltpu.VMEM((_L, _L + 1), f32),
            pltpu.VMEM((_L, _L + 1), f32),
            pltpu.VMEM((_L, _L + 1), f32),
            pltpu.SemaphoreType.DMA,
        ],
    )
    def sc_dots(h_hbm, i0_hbm, i1_hbm, xx_hbm, yy_hbm, xy_hbm,
                i0_v, i1_v, r0_v, r1_v, xx_v, yy_v, xy_v,
                pxx_v, pyy_v, pxy_v, sem):
        wid = lax.axis_index("s") * _NC + lax.axis_index("c")

        def do_chunk(cidx):
            pltpu.sync_copy(i0_hbm.at[cidx], i0_v)
            pltpu.sync_copy(i1_hbm.at[cidx], i1_v)
            cps = []
            for j in range(_BC // _D):
                cps.append(pltpu.async_copy(
                    h_hbm.at[i0_v.at[j]], r0_v.at[pl.ds(j * _D, _D)], sem))
                cps.append(pltpu.async_copy(
                    h_hbm.at[i1_v.at[j]], r1_v.at[pl.ds(j * _D, _D)], sem))
            for cp in cps:
                cp.wait()

            def group_body(g, carry):
                base16 = g * _L
                for e in range(_L):
                    ge = base16 + e
                    px = py = pxy = jnp.zeros((_L,), f32)
                    for k in range(_D // _L):
                        xv = r0_v[ge, pl.ds(k * _L, _L)]
                        yv = r1_v[ge, pl.ds(k * _L, _L)]
                        px = px + xv * xv
                        py = py + yv * yv
                        pxy = pxy + xv * yv
                    pxx_v[e, pl.ds(0, _L)] = px
                    pyy_v[e, pl.ds(0, _L)] = py
                    pxy_v[e, pl.ds(0, _L)] = pxy
                lanes = lax.iota(jnp.int32, _L)
                axx = ayy = axy = jnp.zeros((_L,), f32)
                for k in range(_L):
                    col = jnp.full((_L,), k, jnp.int32)
                    axx = axx + plsc.load_gather(pxx_v, [lanes, col])
                    ayy = ayy + plsc.load_gather(pyy_v, [lanes, col])
                    axy = axy + plsc.load_gather(pxy_v, [lanes, col])
                xx_v[pl.ds(base16, _L)] = axx
                yy_v[pl.ds(base16, _L)] = ayy
                xy_v[pl.ds(base16, _L)] = axy
                return carry

            lax.fori_loop(0, _BC // _L, group_body, 0)
            base = cidx * _BC
            pltpu.sync_copy(xx_v, xx_hbm.at[pl.ds(base, _BC)])
            pltpu.sync_copy(yy_v, yy_hbm.at[pl.ds(base, _BC)])
            pltpu.sync_copy(xy_v, xy_hbm.at[pl.ds(base, _BC)])

        def t_body(t, carry):
            cidx = wid + t * _NW

            @pl.when(cidx < nchunk)
            def _():
                do_chunk(cidx)

            return carry

        lax.fori_loop(0, nt, t_body, 0)

    return sc_dots


def _tail_body(xx_ref, yy_ref, xy_ref, o_ref):
    xx = xx_ref[...]
    yy = yy_ref[...]
    xy = xy_ref[...]
    nx = jnp.maximum(jnp.sqrt(xx), _MIN_NORM)
    ny = jnp.maximum(jnp.sqrt(yy), _MIN_NORM)
    fx = jnp.where(nx > _MAXNORM, _MAXNORM / nx, 1.0)
    fy = jnp.where(ny > _MAXNORM, _MAXNORM / ny, 1.0)
    x2 = fx * fx * xx
    y2 = fy * fy * yy
    pxy = fx * fy * xy
    a = 1.0 - 2.0 * _C * pxy + _C * y2
    b = 1.0 - _C * x2
    num2 = a * a * x2 + b * b * y2 - 2.0 * a * b * pxy
    den = jnp.maximum(1.0 - 2.0 * _C * pxy + _C * _C * x2 * y2, _MIN_NORM)
    sqrt_c = _C ** 0.5
    z = sqrt_c * jnp.sqrt(jnp.maximum(num2, 0.0)) / den
    zc = jnp.clip(z, -1.0 + 1e-7, 1.0 - 1e-7)
    dist_c = 0.5 * jnp.log((1.0 + zc) / (1.0 - zc))
    dist = dist_c * 2.0 / sqrt_c
    sqdist = dist * dist
    o_ref[...] = 1.0 / (jnp.exp((sqdist - _R) / _T) + 1.0)


@functools.lru_cache(maxsize=None)
def _make_tail(n_edges):
    rows = n_edges // _D
    return pl.pallas_call(
        _tail_body,
        out_shape=jax.ShapeDtypeStruct((rows, _D), jnp.float32),
    )


def kernel(h, idx):
    n_nodes, d = h.shape
    n_edges = idx.shape[0]
    assert d == _D
    i0 = idx[:, 0].reshape(-1, _BC // _D, _D)
    i1 = idx[:, 1].reshape(-1, _BC // _D, _D)
    xx, yy, xy = _make_sc_dots(n_nodes, n_edges)(h, i0, i1)
    probs = _make_tail(n_edges)(
        xx.reshape(-1, _D), yy.reshape(-1, _D), xy.reshape(-1, _D))
    return probs.reshape(n_edges)


# speedup vs baseline: 4.3229x; 1.1996x over previous
import functools

import jax
import jax.numpy as jnp
from jax import lax
from jax.experimental import pallas as pl
from jax.experimental.pallas import tpu as pltpu
from jax.experimental.pallas import tpu_sc as plsc

_C = 1.0
_R = 2.0
_T = 1.0
_MIN_NORM = 1e-15
_MAXNORM = (1.0 - 4e-3) / (_C ** 0.5)

_L = 16
_D = 128
_BC = 200
_WB = 10
_NC = 2
_NS = 16
_NW = _NC * _NS
_SPLIT = (104, 96)


@functools.lru_cache(maxsize=None)
def _make_sc_dots(n_nodes, n_edges):
    assert n_edges % (_NW * _BC * _WB) == 0
    cpw = n_edges // (_NW * _BC)
    assert cpw % 2 == 0
    ngrp = -(-_BC // _L)
    mesh = plsc.VectorSubcoreMesh(core_axis_name="c", subcore_axis_name="s")
    f32 = jnp.float32

    @functools.partial(
        pl.kernel,
        mesh=mesh,
        compiler_params=pltpu.CompilerParams(needs_layout_passes=False),
        out_type=[jax.ShapeDtypeStruct((n_edges,), f32)] * 3,
        scratch_types=[
            pltpu.VMEM((_BC,), jnp.int32),
            pltpu.VMEM((_BC,), jnp.int32),
            pltpu.VMEM((_BC,), jnp.int32),
            pltpu.VMEM((_BC,), jnp.int32),
            pltpu.VMEM((_BC, _D), f32),
            pltpu.VMEM((_BC, _D), f32),
            pltpu.VMEM((_BC, _D), f32),
            pltpu.VMEM((_BC, _D), f32),
            pltpu.VMEM((_WB * _BC,), f32),
            pltpu.VMEM((_WB * _BC,), f32),
            pltpu.VMEM((_WB * _BC,), f32),
            pltpu.VMEM((_L, _L + 1), f32),
            pltpu.VMEM((_L, _L + 1), f32),
            pltpu.VMEM((_L, _L + 1), f32),
            pltpu.SemaphoreType.DMA,
            pltpu.SemaphoreType.DMA,
        ],
    )
    def sc_dots(h_hbm, i0_hbm, i1_hbm, xx_hbm, yy_hbm, xy_hbm,
                i0a_v, i0b_v, i1a_v, i1b_v, r0a_v, r0b_v, r1a_v, r1b_v,
                xx_v, yy_v, xy_v, pxx_v, pyy_v, pxy_v, sem0, sem1):
        wid = lax.axis_index("s") * _NC + lax.axis_index("c")
        sems = (sem0, sem1)
        i0_bufs = (i0a_v, i0b_v)
        i1_bufs = (i1a_v, i1b_v)
        r0_bufs = (r0a_v, r0b_v)
        r1_bufs = (r1a_v, r1b_v)

        def copies(b):
            cps = []
            off = 0
            for w in _SPLIT:
                cps.append(pltpu.make_async_copy(
                    h_hbm.at[i0_bufs[b].at[pl.ds(off, w)]],
                    r0_bufs[b].at[pl.ds(off, w)], sems[b]))
                cps.append(pltpu.make_async_copy(
                    h_hbm.at[i1_bufs[b].at[pl.ds(off, w)]],
                    r1_bufs[b].at[pl.ds(off, w)], sems[b]))
                off += w
            return cps

        def fire(b, cidx):
            pltpu.sync_copy(i0_hbm.at[pl.ds(cidx * _BC, _BC)], i0_bufs[b])
            pltpu.sync_copy(i1_hbm.at[pl.ds(cidx * _BC, _BC)], i1_bufs[b])
            for cp in copies(b):
                cp.start()

        def drain(b):
            for cp in copies(b):
                cp.wait()

        def compute(b, slot):
            sbase = (slot % _WB) * _BC
            r0_v = r0_bufs[b]
            r1_v = r1_bufs[b]

            def group_body(g, carry):
                base16 = jnp.minimum(g * _L, _BC - _L)
                for e in range(_L):
                    ge = base16 + e
                    px = py = pxy = jnp.zeros((_L,), f32)
                    for k in range(_D // _L):
                        xv = r0_v[ge, pl.ds(k * _L, _L)]
                        yv = r1_v[ge, pl.ds(k * _L, _L)]
                        px = px + xv * xv
                        py = py + yv * yv
                        pxy = pxy + xv * yv
                    pxx_v[e, pl.ds(0, _L)] = px
                    pyy_v[e, pl.ds(0, _L)] = py
                    pxy_v[e, pl.ds(0, _L)] = pxy
                lanes = lax.iota(jnp.int32, _L)
                axx = ayy = axy = jnp.zeros((_L,), f32)
                for k in range(_L):
                    col = jnp.full((_L,), k, jnp.int32)
                    axx = axx + plsc.load_gather(pxx_v, [lanes, col])
                    ayy = ayy + plsc.load_gather(pyy_v, [lanes, col])
                    axy = axy + plsc.load_gather(pxy_v, [lanes, col])
                dst = sbase + base16
                xx_v[pl.ds(dst, _L)] = axx
                yy_v[pl.ds(dst, _L)] = ayy
                xy_v[pl.ds(dst, _L)] = axy
                return carry

            lax.fori_loop(0, ngrp, group_body, 0)

        def writeback(slot):
            ebase = (wid * cpw + slot - (_WB - 1)) * _BC
            pltpu.sync_copy(xx_v, xx_hbm.at[pl.ds(ebase, _WB * _BC)])
            pltpu.sync_copy(yy_v, yy_hbm.at[pl.ds(ebase, _WB * _BC)])
            pltpu.sync_copy(xy_v, xy_hbm.at[pl.ds(ebase, _WB * _BC)])

        c0 = wid * cpw
        fire(0, c0)

        def t_body(t2, carry):
            for b in range(2):
                slot = t2 * 2 + b
                cidx = c0 + slot

                @pl.when(slot < cpw - 1)
                def _():
                    fire(1 - b, cidx + 1)

                drain(b)
                compute(b, slot)

                @pl.when(slot % _WB == _WB - 1)
                def _():
                    writeback(slot)

            return carry

        lax.fori_loop(0, cpw // 2, t_body, 0)

    return sc_dots


def _tail_body(xx_ref, yy_ref, xy_ref, o_ref):
    xx = xx_ref[...]
    yy = yy_ref[...]
    xy = xy_ref[...]
    nx = jnp.maximum(jnp.sqrt(xx), _MIN_NORM)
    ny = jnp.maximum(jnp.sqrt(yy), _MIN_NORM)
    fx = jnp.where(nx > _MAXNORM, _MAXNORM / nx, 1.0)
    fy = jnp.where(ny > _MAXNORM, _MAXNORM / ny, 1.0)
    x2 = fx * fx * xx
    y2 = fy * fy * yy
    pxy = fx * fy * xy
    a = 1.0 - 2.0 * _C * pxy + _C * y2
    b = 1.0 - _C * x2
    num2 = a * a * x2 + b * b * y2 - 2.0 * a * b * pxy
    den = jnp.maximum(1.0 - 2.0 * _C * pxy + _C * _C * x2 * y2, _MIN_NORM)
    sqrt_c = _C ** 0.5
    z = sqrt_c * jnp.sqrt(jnp.maximum(num2, 0.0)) / den
    zc = jnp.clip(z, -1.0 + 1e-7, 1.0 - 1e-7)
    dist_c = 0.5 * jnp.log((1.0 + zc) / (1.0 - zc))
    dist = dist_c * 2.0 / sqrt_c
    sqdist = dist * dist
    o_ref[...] = 1.0 / (jnp.exp((sqdist - _R) / _T) + 1.0)


@functools.lru_cache(maxsize=None)
def _make_tail(n_edges):
    rows = n_edges // _D
    return pl.pallas_call(
        _tail_body,
        out_shape=jax.ShapeDtypeStruct((rows, _D), jnp.float32),
    )


def kernel(h, idx):
    n_nodes, d = h.shape
    n_edges = idx.shape[0]
    assert d == _D
    i0 = idx[:, 0]
    i1 = idx[:, 1]
    xx, yy, xy = _make_sc_dots(n_nodes, n_edges)(h, i0, i1)
    probs = _make_tail(n_edges)(
        xx.reshape(-1, _D), yy.reshape(-1, _D), xy.reshape(-1, _D))
    return probs.reshape(n_edges)


# speedup vs baseline: 6.5355x; 1.5118x over previous
import functools

import jax
import jax.numpy as jnp
from jax import lax
from jax.experimental import pallas as pl
from jax.experimental.pallas import tpu as pltpu
from jax.experimental.pallas import tpu_sc as plsc

_C = 1.0
_R = 2.0
_T = 1.0
_MIN_NORM = 1e-15
_MAXNORM = (1.0 - 4e-3) / (_C ** 0.5)

_L = 16
_D = 128
_BC = 200
_WB = 10
_NB = 3
_NC = 2
_NS = 16
_NW = _NC * _NS
_SPLIT = (104, 96)


@functools.lru_cache(maxsize=None)
def _make_sc_dots(n_nodes, n_edges):
    assert n_edges % (_NW * _BC * _WB) == 0
    cpw = n_edges // (_NW * _BC)
    nt3 = (cpw - 2) // _NB
    tail_slots = list(range(nt3 * _NB, cpw))
    ngrp = -(-_BC // _L)
    mesh = plsc.VectorSubcoreMesh(core_axis_name="c", subcore_axis_name="s")
    f32 = jnp.float32

    @functools.partial(
        pl.kernel,
        mesh=mesh,
        compiler_params=pltpu.CompilerParams(needs_layout_passes=False),
        out_type=[jax.ShapeDtypeStruct((n_edges,), f32)] * 3,
        scratch_types=[
            pltpu.VMEM((_BC,), jnp.int32),
            pltpu.VMEM((_BC,), jnp.int32),
            pltpu.VMEM((_BC,), jnp.int32),
            pltpu.VMEM((_BC,), jnp.int32),
            pltpu.VMEM((_BC,), jnp.int32),
            pltpu.VMEM((_BC,), jnp.int32),
            pltpu.VMEM((_BC, _D), f32),
            pltpu.VMEM((_BC, _D), f32),
            pltpu.VMEM((_BC, _D), f32),
            pltpu.VMEM((n_nodes,), f32),
            pltpu.VMEM((_WB * _BC,), f32),
            pltpu.VMEM((_WB * _BC,), f32),
            pltpu.VMEM((_WB * _BC,), f32),
            pltpu.VMEM((_L, _L + 1), f32),
            pltpu.SemaphoreType.DMA,
            pltpu.SemaphoreType.DMA,
            pltpu.SemaphoreType.DMA,
        ],
    )
    def sc_dots(h_hbm, i0_hbm, i1_hbm, sqn_hbm, xx_hbm, yy_hbm, xy_hbm,
                i0a, i0b, i0c, i1a, i1b, i1c, rsa, rsb, rsc, sqn_v,
                xx_v, yy_v, xy_v, ps_v, sem0, sem1, sem2):
        wid = lax.axis_index("s") * _NC + lax.axis_index("c")
        sems = (sem0, sem1, sem2)
        i0_bufs = (i0a, i0b, i0c)
        i1_bufs = (i1a, i1b, i1c)
        rs_bufs = (rsa, rsb, rsc)
        c0 = wid * cpw

        pltpu.sync_copy(sqn_hbm, sqn_v)

        def copies(b, side):
            idx = (i0_bufs if side == 0 else i1_bufs)[b]
            cps = []
            off = 0
            for w in _SPLIT:
                cps.append(pltpu.make_async_copy(
                    h_hbm.at[idx.at[pl.ds(off, w)]],
                    rs_bufs[b].at[pl.ds(off, w)], sems[b]))
                off += w
            return cps

        def fire_w(b, slot):
            cidx = c0 + slot
            pltpu.sync_copy(i0_hbm.at[pl.ds(cidx * _BC, _BC)], i0_bufs[b])
            pltpu.sync_copy(i1_hbm.at[pl.ds(cidx * _BC, _BC)], i1_bufs[b])
            for cp in copies(b, 0):
                cp.start()

        def wait(b, side):
            for cp in copies(b, side):
                cp.wait()

        def fire_a(b):
            for cp in copies(b, 1):
                cp.start(add=True)

        def compute(b, slot):
            sbase = (slot % _WB) * _BC
            rs_v = rs_bufs[b]

            def group_body(g, carry):
                base16 = jnp.minimum(g * _L, _BC - _L)
                for e in range(_L):
                    ge = base16 + e
                    ps = jnp.zeros((_L,), f32)
                    for k in range(_D // _L):
                        sv = rs_v[ge, pl.ds(k * _L, _L)]
                        ps = ps + sv * sv
                    ps_v[e, pl.ds(0, _L)] = ps
                lanes = lax.iota(jnp.int32, _L)
                s2 = jnp.zeros((_L,), f32)
                for k in range(_L):
                    col = jnp.full((_L,), k, jnp.int32)
                    s2 = s2 + plsc.load_gather(ps_v, [lanes, col])
                i0l = i0_bufs[b][pl.ds(base16, _L)]
                i1l = i1_bufs[b][pl.ds(base16, _L)]
                xx = plsc.load_gather(sqn_v, [i0l])
                yy = plsc.load_gather(sqn_v, [i1l])
                xy = 0.5 * (s2 - xx - yy)
                dst = sbase + base16
                xx_v[pl.ds(dst, _L)] = xx
                yy_v[pl.ds(dst, _L)] = yy
                xy_v[pl.ds(dst, _L)] = xy
                return carry

            lax.fori_loop(0, ngrp, group_body, 0)

            @pl.when(slot % _WB == _WB - 1)
            def _():
                ebase = (c0 + slot - (_WB - 1)) * _BC
                pltpu.sync_copy(xx_v, xx_hbm.at[pl.ds(ebase, _WB * _BC)])
                pltpu.sync_copy(yy_v, yy_hbm.at[pl.ds(ebase, _WB * _BC)])
                pltpu.sync_copy(xy_v, xy_hbm.at[pl.ds(ebase, _WB * _BC)])

        def step(slot, b):
            bw = (b + 2) % _NB
            ba = (b + 1) % _NB

            @pl.when(slot + 2 < cpw)
            def _():
                fire_w(bw, slot + 2)

            @pl.when(slot + 1 < cpw)
            def _():
                wait(ba, 0)
                fire_a(ba)

            wait(b, 1)
            compute(b, slot)

        fire_w(0, 0)
        wait(0, 0)
        fire_a(0)
        fire_w(1, 1)

        def t_body(t3, carry):
            for b in range(_NB):
                step(t3 * _NB + b, b)
            return carry

        lax.fori_loop(0, nt3, t_body, 0)
        for slot in tail_slots:
            step(slot, slot % _NB)

    return sc_dots


def _sqn_body(h_ref, o_ref):
    x = h_ref[...]
    o_ref[...] = jnp.sum(x * x, axis=1, keepdims=True)


@functools.lru_cache(maxsize=None)
def _make_sqn(n_nodes):
    return pl.pallas_call(
        _sqn_body,
        out_shape=jax.ShapeDtypeStruct((n_nodes, 1), jnp.float32),
    )


def _tail_body(xx_ref, yy_ref, xy_ref, o_ref):
    xx = xx_ref[...]
    yy = yy_ref[...]
    xy = xy_ref[...]
    nx = jnp.maximum(jnp.sqrt(xx), _MIN_NORM)
    ny = jnp.maximum(jnp.sqrt(yy), _MIN_NORM)
    fx = jnp.where(nx > _MAXNORM, _MAXNORM / nx, 1.0)
    fy = jnp.where(ny > _MAXNORM, _MAXNORM / ny, 1.0)
    x2 = fx * fx * xx
    y2 = fy * fy * yy
    pxy = fx * fy * xy
    a = 1.0 - 2.0 * _C * pxy + _C * y2
    b = 1.0 - _C * x2
    num2 = a * a * x2 + b * b * y2 - 2.0 * a * b * pxy
    den = jnp.maximum(1.0 - 2.0 * _C * pxy + _C * _C * x2 * y2, _MIN_NORM)
    sqrt_c = _C ** 0.5
    z = sqrt_c * jnp.sqrt(jnp.maximum(num2, 0.0)) / den
    zc = jnp.clip(z, -1.0 + 1e-7, 1.0 - 1e-7)
    dist_c = 0.5 * jnp.log((1.0 + zc) / (1.0 - zc))
    dist = dist_c * 2.0 / sqrt_c
    sqdist = dist * dist
    o_ref[...] = 1.0 / (jnp.exp((sqdist - _R) / _T) + 1.0)


@functools.lru_cache(maxsize=None)
def _make_tail(n_edges):
    rows = n_edges // _D
    return pl.pallas_call(
        _tail_body,
        out_shape=jax.ShapeDtypeStruct((rows, _D), jnp.float32),
    )


def kernel(h, idx):
    n_nodes, d = h.shape
    n_edges = idx.shape[0]
    assert d == _D
    i0 = idx[:, 0]
    i1 = idx[:, 1]
    sqn = _make_sqn(n_nodes)(h).reshape(n_nodes)
    xx, yy, xy = _make_sc_dots(n_nodes, n_edges)(h, i0, i1, sqn)
    probs = _make_tail(n_edges)(
        xx.reshape(-1, _D), yy.reshape(-1, _D), xy.reshape(-1, _D))
    return probs.reshape(n_edges)
